# Initial kernel scaffold; baseline (speedup 1.0000x reference)
import jax
import jax.numpy as jnp
from jax.experimental import pallas as pl


def kernel(x, edge_index, W0, b0, W1, b1, W2, b2, W3, b3):
    raise NotImplementedError("write your pallas kernel here")


# speedup vs baseline: 5.5481x; 5.5481x over previous
import functools

import jax
import jax.numpy as jnp
from jax import lax
from jax.experimental import pallas as pl
from jax.experimental.pallas import tpu as pltpu
from jax.experimental.pallas import tpu_sc as plsc

N_NODES = 10000
NP = 10240
E_EDGES = 160000
EP = 163840
N_TILES = 16
TB = 80
TBH = TB // 2
ROWS_PER_TILE = NP // N_TILES
GRID_R = 8
RBLK = NP // GRID_R
C = 128


def _make_sc_agg(num_chunks: int, gather_g: bool = True):
    edge_split = num_chunks == 1
    chunks_per_core = max(num_chunks // 2, 1)
    n_out = 2 if edge_split else num_chunks
    mesh = plsc.VectorSubcoreMesh(core_axis_name="c", subcore_axis_name="s")
    out_type = [jax.ShapeDtypeStruct((NP, C), jnp.float32)
                for _ in range(n_out)]
    scratch = [
        pltpu.VMEM((TB, 128), jnp.int32),
        pltpu.VMEM((TB, 128), jnp.int32),
        pltpu.VMEM((128, C), jnp.float32),
        pltpu.VMEM_SHARED((NP, C), jnp.float32),
        pltpu.SemaphoreType.DMA,
    ]

    @functools.partial(pl.kernel, out_type=out_type, mesh=mesh,
                       scratch_types=scratch)
    def agg(*refs):
        src_hbm, dst_hbm = refs[0], refs[1]
        g_refs = refs[2:2 + num_chunks]
        out_refs = refs[2 + num_chunks:2 + num_chunks + n_out]
        src_vm, dst_vm, buf, acc, sem = refs[2 + num_chunks + n_out:]

        cid = lax.axis_index("c")
        sid = lax.axis_index("s")
        r0 = sid * ROWS_PER_TILE

        def process(g_ref, out_ref, lo, hi):
            pltpu.sync_copy(g_ref.at[pl.ds(r0, ROWS_PER_TILE)],
                            acc.at[pl.ds(r0, ROWS_PER_TILE)])
            if not gather_g:
                pltpu.sync_copy(g_ref.at[pl.ds(0, 128)], buf)
            plsc.subcore_barrier()

            def step(j, carry):
                if gather_g:
                    pltpu.async_copy(g_ref.at[src_vm.at[j]], buf, sem).wait()
                pltpu.sync_copy(buf, acc.at[dst_vm.at[j]], add=True)
                return carry

            lax.fori_loop(lo, hi, step, 0)
            plsc.subcore_barrier()
            pltpu.sync_copy(acc.at[pl.ds(r0, ROWS_PER_TILE)],
                            out_ref.at[pl.ds(r0, ROWS_PER_TILE)])
            plsc.subcore_barrier()

        if edge_split:
            if gather_g:
                pltpu.sync_copy(src_hbm.at[sid], src_vm)
            pltpu.sync_copy(dst_hbm.at[sid], dst_vm)

            @pl.when(cid == 0)
            def _():
                process(g_refs[0], out_refs[0], 0, TBH)

            @pl.when(cid == 1)
            def _():
                process(g_refs[0], out_refs[1], TBH, TB)
        else:
            pltpu.sync_copy(src_hbm.at[sid], src_vm)
            pltpu.sync_copy(dst_hbm.at[sid], dst_vm)

            @pl.when(cid == 0)
            def _():
                for k in range(chunks_per_core):
                    process(g_refs[k], out_refs[k], 0, TB)

            @pl.when(cid == 1)
            def _():
                for k in range(chunks_per_core, num_chunks):
                    process(g_refs[k], out_refs[k], 0, TB)

    return agg


_sc_deg = _make_sc_agg(1, gather_g=False)
_sc_agg_1 = _make_sc_agg(1)
_sc_agg_2 = _make_sc_agg(2)
_sc_agg_4 = _make_sc_agg(4)


def _row_spec(cols):
    return pl.BlockSpec((RBLK, cols), lambda i: (i, 0))


def _full_spec(shape):
    return pl.BlockSpec(shape, lambda i: (0,) * len(shape))


def _tc_scale_x(x_ref, dega_ref, degb_ref, d_ref, ga_ref, gb_ref):
    deg = dega_ref[:, 0:1] + degb_ref[:, 0:1] - 1.0
    d = lax.rsqrt(deg)
    d_ref[...] = d
    g = d * x_ref[...]
    ga_ref[...] = g[:, :128]
    gb_ref[...] = g[:, 128:]


def _tc_layer0(sa, sb, d_ref, w0, b0, w1, o0, o1, o2, o3):
    s = jnp.concatenate([sa[...], sb[...]], axis=1)
    d = d_ref[...]
    h = jnp.maximum(
        jnp.dot(d * s, w0[...], preferred_element_type=jnp.float32) + b0[...],
        0.0)
    g = d * jnp.dot(h, w1[...], preferred_element_type=jnp.float32)
    o0[...] = g[:, 0:128]
    o1[...] = g[:, 128:256]
    o2[...] = g[:, 256:384]
    o3[...] = g[:, 384:512]


def _tc_mid(sa, sb, sc, sd, d_ref, b_ref, w_ref, o0, o1, o2, o3):
    s = jnp.concatenate([sa[...], sb[...], sc[...], sd[...]], axis=1)
    d = d_ref[...]
    h = jnp.maximum(d * s + b_ref[...], 0.0)
    g = d * jnp.dot(h, w_ref[...], preferred_element_type=jnp.float32)
    o0[...] = g[:, 0:128]
    o1[...] = g[:, 128:256]
    o2[...] = g[:, 256:384]
    o3[...] = g[:, 384:512]


def _tc_last(sa, sb, sc, sd, d_ref, b_ref, w_ref, o_ref):
    s = jnp.concatenate([sa[...], sb[...], sc[...], sd[...]], axis=1)
    d = d_ref[...]
    h = jnp.maximum(d * s + b_ref[...], 0.0)
    o_ref[...] = d * jnp.dot(h, w_ref[...], preferred_element_type=jnp.float32)


def _tc_final(s3a_ref, s3b_ref, g3_ref, d_ref, b_ref, o_ref):
    s = s3a_ref[...] + s3b_ref[...] - g3_ref[...]
    o_ref[...] = d_ref[...] * s + b_ref[...]


def kernel(x, edge_index, W0, b0, W1, b1, W2, b2, W3, b3):
    f32 = jnp.float32
    src = edge_index[0].astype(jnp.int32)
    dst = edge_index[1].astype(jnp.int32)
    pad_e = EP - E_EDGES
    dummy = jnp.full((pad_e,), N_NODES, dtype=jnp.int32)
    src3 = jnp.concatenate([src, dummy]).reshape(N_TILES, TB, 128)
    dst3 = jnp.concatenate([dst, dummy]).reshape(N_TILES, TB, 128)

    x_pad = jnp.zeros((NP, 256), f32).at[:N_NODES].set(x)
    ones_c0 = jnp.zeros((NP, C), f32).at[:, 0].set(1.0)
    w3p = jnp.zeros((512, C), f32).at[:, :4].set(W3)
    b3p = jnp.zeros((1, C), f32).at[0, :4].set(b3)
    b0r = b0.reshape(1, 512)
    b1r = b1.reshape(1, 512)
    b2r = b2.reshape(1, 512)

    dega, degb = _sc_deg(src3, dst3, ones_c0)

    d_col, g0a, g0b = pl.pallas_call(
        _tc_scale_x,
        grid=(GRID_R,),
        in_specs=[_row_spec(256), _row_spec(C), _row_spec(C)],
        out_specs=[_row_spec(1), _row_spec(128), _row_spec(128)],
        out_shape=[jax.ShapeDtypeStruct((NP, 1), f32),
                   jax.ShapeDtypeStruct((NP, 128), f32),
                   jax.ShapeDtypeStruct((NP, 128), f32)],
    )(x_pad, dega, degb)
    s0 = _sc_agg_2(src3, dst3, g0a, g0b)

    g1 = pl.pallas_call(
        _tc_layer0,
        grid=(GRID_R,),
        in_specs=[_row_spec(128), _row_spec(128), _row_spec(1),
                  _full_spec((256, 512)), _full_spec((1, 512)),
                  _full_spec((512, 512))],
        out_specs=[_row_spec(128)] * 4,
        out_shape=[jax.ShapeDtypeStruct((NP, 128), f32)] * 4,
    )(s0[0], s0[1], d_col, W0, b0r, W1)
    s1 = _sc_agg_4(src3, dst3, *g1)

    g2 = pl.pallas_call(
        _tc_mid,
        grid=(GRID_R,),
        in_specs=[_row_spec(128)] * 4 + [_row_spec(1),
                  _full_spec((1, 512)), _full_spec((512, 512))],
        out_specs=[_row_spec(128)] * 4,
        out_shape=[jax.ShapeDtypeStruct((NP, 128), f32)] * 4,
    )(*s1, d_col, b1r, W2)
    s2 = _sc_agg_4(src3, dst3, *g2)

    g3 = pl.pallas_call(
        _tc_last,
        grid=(GRID_R,),
        in_specs=[_row_spec(128)] * 4 + [_row_spec(1),
                  _full_spec((1, 512)), _full_spec((512, C))],
        out_specs=_row_spec(C),
        out_shape=jax.ShapeDtypeStruct((NP, C), f32),
    )(*s2, d_col, b2r, w3p)
    s3a, s3b = _sc_agg_1(src3, dst3, g3)

    out = pl.pallas_call(
        _tc_final,
        grid=(GRID_R,),
        in_specs=[_row_spec(C), _row_spec(C), _row_spec(C), _row_spec(1),
                  _full_spec((1, C))],
        out_specs=_row_spec(C),
        out_shape=jax.ShapeDtypeStruct((NP, C), f32),
    )(s3a, s3b, g3, d_col, b3p)

    return out[:N_NODES, :4]


# speedup vs baseline: 6.6462x; 1.1979x over previous
import functools

import jax
import jax.numpy as jnp
from jax import lax
from jax.experimental import pallas as pl
from jax.experimental.pallas import tpu as pltpu
from jax.experimental.pallas import tpu_sc as plsc

N_NODES = 10000
NP = 10240
E_EDGES = 160000
EP = 163840
N_TILES = 16
BATCH = 128
TB = 80
WIN = 40
EPT = TB * BATCH
ROWS_PER_TILE = NP // N_TILES
GRID_R = 8
RBLK = NP // GRID_R
C = 128


def _make_sc_agg(num_chunks: int, gather_g: bool = True):
    edge_split = num_chunks == 1
    chunks_per_core = max(num_chunks // 2, 1)
    n_out = 2 if edge_split else num_chunks
    nbuf = 2
    mesh = plsc.VectorSubcoreMesh(core_axis_name="c", subcore_axis_name="s")
    out_type = [jax.ShapeDtypeStruct((NP, C), jnp.float32)
                for _ in range(n_out)]
    scratch = [
        pltpu.VMEM((EPT,), jnp.int32),
        pltpu.VMEM((WIN, BATCH), jnp.int32),
        pltpu.VMEM((nbuf, BATCH, C), jnp.float32),
        pltpu.VMEM_SHARED((NP, C), jnp.float32),
        pltpu.SemaphoreType.DMA((nbuf,)),
        pltpu.SemaphoreType.DMA((nbuf,)),
    ]

    @functools.partial(pl.kernel, out_type=out_type, mesh=mesh,
                       scratch_types=scratch)
    def agg(*refs):
        src_hbm, dst_hbm = refs[0], refs[1]
        g_refs = refs[2:2 + num_chunks]
        out_refs = refs[2 + num_chunks:2 + num_chunks + n_out]
        src_vm, dst_vm, buf, acc, gsem, ssem = refs[2 + num_chunks + n_out:]

        cid = lax.axis_index("c")
        sid = lax.axis_index("s")
        r0 = sid * ROWS_PER_TILE

        def ring(g_ref, lo, n, dst_base):
            def gather(j, b):
                return pltpu.async_copy(
                    g_ref.at[src_vm.at[pl.ds(j * BATCH, BATCH)]],
                    buf.at[b], gsem.at[b])

            def scat(j, b):
                return pltpu.async_copy(buf.at[b],
                                        acc.at[dst_vm.at[j - dst_base]],
                                        ssem.at[b], add=True)

            for b in range(nbuf):
                gather(lo + b, b)

            def round_(r, carry):
                jbase = lo + r * nbuf
                for b in range(nbuf):
                    j = jbase + b
                    pltpu.make_async_copy(
                        g_ref.at[src_vm.at[pl.ds(0, BATCH)]],
                        buf.at[b], gsem.at[b]).wait()
                    scat(j, b)

                    @pl.when(j + nbuf < lo + n)
                    def _():
                        pltpu.make_async_copy(
                            buf.at[b], acc.at[dst_vm.at[0]],
                            ssem.at[b]).wait()
                        gather(j + nbuf, b)
                return carry

            lax.fori_loop(0, n // nbuf, round_, 0)
            for b in range(nbuf):
                pltpu.make_async_copy(buf.at[b], acc.at[dst_vm.at[0]],
                                      ssem.at[b]).wait()

        def scatter_only(lo, n, dst_base):
            def round_(r, carry):
                jbase = lo + r * nbuf
                for b in range(nbuf):
                    pltpu.async_copy(buf.at[b],
                                     acc.at[dst_vm.at[jbase + b - dst_base]],
                                     ssem.at[b], add=True)
                for b in range(nbuf):
                    pltpu.make_async_copy(buf.at[b], acc.at[dst_vm.at[0]],
                                          ssem.at[b]).wait()
                return carry

            lax.fori_loop(0, n // nbuf, round_, 0)

        def process(g_ref, out_ref, lo, n_batches):
            pltpu.sync_copy(g_ref.at[pl.ds(r0, ROWS_PER_TILE)],
                            acc.at[pl.ds(r0, ROWS_PER_TILE)])
            if not gather_g:
                for b in range(nbuf):
                    pltpu.sync_copy(g_ref.at[pl.ds(0, BATCH)], buf.at[b])
            plsc.subcore_barrier()

            for w in range(n_batches // WIN):
                wlo = lo + w * WIN
                pltpu.sync_copy(dst_hbm.at[sid].at[pl.ds(wlo, WIN)], dst_vm)
                if gather_g:
                    ring(g_ref, wlo, WIN, wlo)
                else:
                    scatter_only(wlo, WIN, wlo)

            plsc.subcore_barrier()
            pltpu.sync_copy(acc.at[pl.ds(r0, ROWS_PER_TILE)],
                            out_ref.at[pl.ds(r0, ROWS_PER_TILE)])
            plsc.subcore_barrier()

        if gather_g:
            pltpu.sync_copy(src_hbm.at[sid], src_vm)

        if edge_split:
            @pl.when(cid == 0)
            def _():
                process(g_refs[0], out_refs[0], 0, TB // 2)

            @pl.when(cid == 1)
            def _():
                process(g_refs[0], out_refs[1], TB // 2, TB // 2)
        else:
            @pl.when(cid == 0)
            def _():
                for k in range(chunks_per_core):
                    process(g_refs[k], out_refs[k], 0, TB)

            @pl.when(cid == 1)
            def _():
                for k in range(chunks_per_core, num_chunks):
                    process(g_refs[k], out_refs[k], 0, TB)

    return agg


_sc_deg = _make_sc_agg(1, gather_g=False)
_sc_agg_1 = _make_sc_agg(1)
_sc_agg_2 = _make_sc_agg(2)
_sc_agg_4 = _make_sc_agg(4)


def _row_spec(cols):
    return pl.BlockSpec((RBLK, cols), lambda i: (i, 0))


def _full_spec(shape):
    return pl.BlockSpec(shape, lambda i: (0,) * len(shape))


def _tc_scale_x(x_ref, dega_ref, degb_ref, d_ref, ga_ref, gb_ref):
    deg = dega_ref[:, 0:1] + degb_ref[:, 0:1] - 1.0
    d = lax.rsqrt(deg)
    d_ref[...] = d
    g = d * x_ref[...]
    ga_ref[...] = g[:, :128]
    gb_ref[...] = g[:, 128:]


def _tc_layer0(sa, sb, d_ref, w0, b0, w1, o0, o1, o2, o3):
    s = jnp.concatenate([sa[...], sb[...]], axis=1)
    d = d_ref[...]
    h = jnp.maximum(
        jnp.dot(d * s, w0[...], preferred_element_type=jnp.float32) + b0[...],
        0.0)
    g = d * jnp.dot(h, w1[...], preferred_element_type=jnp.float32)
    o0[...] = g[:, 0:128]
    o1[...] = g[:, 128:256]
    o2[...] = g[:, 256:384]
    o3[...] = g[:, 384:512]


def _tc_mid(sa, sb, sc, sd, d_ref, b_ref, w_ref, o0, o1, o2, o3):
    s = jnp.concatenate([sa[...], sb[...], sc[...], sd[...]], axis=1)
    d = d_ref[...]
    h = jnp.maximum(d * s + b_ref[...], 0.0)
    g = d * jnp.dot(h, w_ref[...], preferred_element_type=jnp.float32)
    o0[...] = g[:, 0:128]
    o1[...] = g[:, 128:256]
    o2[...] = g[:, 256:384]
    o3[...] = g[:, 384:512]


def _tc_last(sa, sb, sc, sd, d_ref, b_ref, w_ref, o_ref):
    s = jnp.concatenate([sa[...], sb[...], sc[...], sd[...]], axis=1)
    d = d_ref[...]
    h = jnp.maximum(d * s + b_ref[...], 0.0)
    o_ref[...] = d * jnp.dot(h, w_ref[...], preferred_element_type=jnp.float32)


def _tc_final(s3a_ref, s3b_ref, g3_ref, d_ref, b_ref, o_ref):
    s = s3a_ref[...] + s3b_ref[...] - g3_ref[...]
    o_ref[...] = d_ref[...] * s + b_ref[...]


def kernel(x, edge_index, W0, b0, W1, b1, W2, b2, W3, b3):
    f32 = jnp.float32
    src = edge_index[0].astype(jnp.int32)
    dst = edge_index[1].astype(jnp.int32)
    pad_e = EP - E_EDGES
    dummy = jnp.full((pad_e,), N_NODES, dtype=jnp.int32)
    src3 = jnp.concatenate([src, dummy]).reshape(N_TILES, EPT)
    dst3 = jnp.concatenate([dst, dummy]).reshape(N_TILES, TB, BATCH)

    x_pad = jnp.zeros((NP, 256), f32).at[:N_NODES].set(x)
    ones_c0 = jnp.zeros((NP, C), f32).at[:, 0].set(1.0)
    w3p = jnp.zeros((512, C), f32).at[:, :4].set(W3)
    b3p = jnp.zeros((1, C), f32).at[0, :4].set(b3)
    b0r = b0.reshape(1, 512)
    b1r = b1.reshape(1, 512)
    b2r = b2.reshape(1, 512)

    dega, degb = _sc_deg(src3, dst3, ones_c0)

    d_col, g0a, g0b = pl.pallas_call(
        _tc_scale_x,
        grid=(GRID_R,),
        in_specs=[_row_spec(256), _row_spec(C), _row_spec(C)],
        out_specs=[_row_spec(1), _row_spec(128), _row_spec(128)],
        out_shape=[jax.ShapeDtypeStruct((NP, 1), f32),
                   jax.ShapeDtypeStruct((NP, 128), f32),
                   jax.ShapeDtypeStruct((NP, 128), f32)],
    )(x_pad, dega, degb)
    s0 = _sc_agg_2(src3, dst3, g0a, g0b)

    g1 = pl.pallas_call(
        _tc_layer0,
        grid=(GRID_R,),
        in_specs=[_row_spec(128), _row_spec(128), _row_spec(1),
                  _full_spec((256, 512)), _full_spec((1, 512)),
                  _full_spec((512, 512))],
        out_specs=[_row_spec(128)] * 4,
        out_shape=[jax.ShapeDtypeStruct((NP, 128), f32)] * 4,
    )(s0[0], s0[1], d_col, W0, b0r, W1)
    s1 = _sc_agg_4(src3, dst3, *g1)

    g2 = pl.pallas_call(
        _tc_mid,
        grid=(GRID_R,),
        in_specs=[_row_spec(128)] * 4 + [_row_spec(1),
                  _full_spec((1, 512)), _full_spec((512, 512))],
        out_specs=[_row_spec(128)] * 4,
        out_shape=[jax.ShapeDtypeStruct((NP, 128), f32)] * 4,
    )(*s1, d_col, b1r, W2)
    s2 = _sc_agg_4(src3, dst3, *g2)

    g3 = pl.pallas_call(
        _tc_last,
        grid=(GRID_R,),
        in_specs=[_row_spec(128)] * 4 + [_row_spec(1),
                  _full_spec((1, 512)), _full_spec((512, C))],
        out_specs=_row_spec(C),
        out_shape=jax.ShapeDtypeStruct((NP, C), f32),
    )(*s2, d_col, b2r, w3p)
    s3a, s3b = _sc_agg_1(src3, dst3, g3)

    out = pl.pallas_call(
        _tc_final,
        grid=(GRID_R,),
        in_specs=[_row_spec(C), _row_spec(C), _row_spec(C), _row_spec(1),
                  _full_spec((1, C))],
        out_specs=_row_spec(C),
        out_shape=jax.ShapeDtypeStruct((NP, C), f32),
    )(s3a, s3b, g3, d_col, b3p)

    return out[:N_NODES, :4]
